# baseline (speedup 1.0000x reference)
import functools

import jax
import jax.numpy as jnp
from jax import lax
from jax.experimental import pallas as pl
from jax.experimental.pallas import tpu as pltpu
from jax.experimental.pallas import tpu_sc as plsc

ALPHA = 0.5
ITERS = 10
NC = 2
NS = 16
NW = NC * NS
W = 128
LANES = 16
NBUF = 2
CH = 8


def _sc_step_body(n_win, rows_per_sub, d_reg,
                  out_hbm, col_hbm, row_hbm, wa_hbm, parts_hbm,
                  acc, colv, rc0, rc1, wc0, wc1,
                  g0, g1, gs0, gs1, ss0, ss1, rs0, rs1):
  gbufs = (g0, g1)
  gsems = (gs0, gs1)
  ssems = (ss0, ss1)
  rcs = (rc0, rc1)
  wcs = (wc0, wc1)
  rsems = (rs0, rs1)
  c = lax.axis_index("c")
  s = lax.axis_index("s")
  widx = c * NS + s
  n_chunk = n_win // CH

  def gather_start(j, t):
    pltpu.async_copy(out_hbm.at[colv.at[j]], gbufs[t], gsems[t])

  def gather_wait(j, t):
    pltpu.make_async_copy(out_hbm.at[colv.at[j]], gbufs[t], gsems[t]).wait()

  def scatter_start(j, jc, t, tc):
    pltpu.async_copy(gbufs[t], acc.at[rcs[tc].at[jc]], ssems[t], add=True)

  def scatter_wait(j, jc, t, tc):
    pltpu.make_async_copy(
        gbufs[t], acc.at[rcs[tc].at[jc]], ssems[t]).wait()

  def chunk_start(cc, tc):
    pltpu.async_copy(row_hbm.at[widx].at[pl.ds(cc * CH, CH)], rcs[tc],
                     rsems[tc])
    pltpu.async_copy(wa_hbm.at[widx].at[pl.ds(cc * CH, CH)], wcs[tc],
                     rsems[tc])

  def chunk_wait(cc, tc):
    pltpu.make_async_copy(row_hbm.at[widx].at[pl.ds(cc * CH, CH)], rcs[tc],
                          rsems[tc]).wait()
    pltpu.make_async_copy(wa_hbm.at[widx].at[pl.ds(cc * CH, CH)], wcs[tc],
                          rsems[tc]).wait()

  pltpu.sync_copy(col_hbm.at[widx], colv)
  gather_start(0, 0)
  chunk_start(0, 0)

  zv = jnp.zeros((LANES,), jnp.float32)

  @pl.loop(0, W)
  def _zrow(r):
    for k in range(d_reg):
      g1[r, pl.ds(k * LANES, LANES)] = zv

  base = s * rows_per_sub
  nfull = rows_per_sub // W
  rem = rows_per_sub - nfull * W
  zchunks = [(base + z * W, W) for z in range(nfull)]
  if rem:
    zchunks.append((base + nfull * W, rem))
  for off, sz in zchunks:
    pltpu.async_copy(g1.at[pl.ds(0, sz)], acc.at[pl.ds(off, sz)], ss1)
  chunk_wait(0, 0)
  for off, sz in zchunks:
    pltpu.make_async_copy(g1.at[pl.ds(0, sz)], acc.at[pl.ds(off, sz)],
                          ss1).wait()
  plsc.subcore_barrier()

  nblk = W // LANES

  def scale_blocks(jc, t, tc, b_lo, b_hi):
    @pl.loop(b_lo, b_hi)
    def _blk(b):
      wv = wcs[tc][jc, pl.ds(b * LANES, LANES)]
      for e16 in range(LANES):
        ws = wv.at[jnp.full((LANES,), e16, jnp.int32)].get(
            mode="promise_in_bounds")
        e = b * LANES + e16
        for k in range(d_reg):
          sl = pl.ds(k * LANES, LANES)
          gbufs[t][e, sl] = gbufs[t][e, sl] * ws

  @pl.loop(0, n_chunk, step=2)
  def _chunkpair(cc0):
    for ccs in range(2):
      cc = cc0 + ccs
      tcur = ccs
      for jcs in range(CH):
        j = cc * CH + jcs
        t = jcs % NBUF
        to = 1 - t
        gather_wait(j, t)

        @pl.when(j >= 1)
        def _():
          if jcs == 0:
            scatter_wait(j - 1, CH - 1, to, 1 - tcur)
          else:
            scatter_wait(j - 1, jcs - 1, to, tcur)

        @pl.when(j + 1 < n_win)
        def _():
          gather_start(j + 1, to)

        if jcs == 1:
          @pl.when(cc + 1 < n_chunk)
          def _():
            chunk_start(cc + 1, 1 - tcur)

        scale_blocks(jcs, t, tcur, 0, nblk)
        scatter_start(j, jcs, t, tcur)

      @pl.when(cc + 1 < n_chunk)
      def _():
        chunk_wait(cc + 1, 1 - tcur)

  scatter_wait(n_win - 1, CH - 1, (n_win - 1) % NBUF, (n_chunk - 1) % 2)

  plsc.subcore_barrier()

  nz = len(zchunks)

  def ro_in(i):
    off, sz = zchunks[i]
    return (acc.at[pl.ds(off, sz)], gbufs[i % 2].at[pl.ds(0, sz)],
            gsems[i % 2])

  def ro_out(i):
    off, sz = zchunks[i]
    return (gbufs[i % 2].at[pl.ds(0, sz)], parts_hbm.at[c].at[pl.ds(off, sz)],
            ssems[i % 2])

  pltpu.async_copy(*ro_in(0))
  for i in range(nz):
    pltpu.make_async_copy(*ro_in(i)).wait()
    if i + 1 < nz:
      if i >= 1:
        pltpu.make_async_copy(*ro_out(i - 1)).wait()
      pltpu.async_copy(*ro_in(i + 1))
    pltpu.async_copy(*ro_out(i))
  pltpu.make_async_copy(*ro_out(nz - 1)).wait()
  if nz > 1:
    pltpu.make_async_copy(*ro_out(nz - 2)).wait()


def _prep_body(x_ref, w_ref, res_ref, wa_ref):
  res_ref[...] = x_ref[...] * (1.0 - ALPHA)
  wa_ref[...] = w_ref[...] * ALPHA


def _combine_body(p_ref, res_ref, o_ref):
  o_ref[...] = p_ref[0] + p_ref[1] + res_ref[...]


def kernel(x, edge_index, edge_weight):
  n_nodes, d = x.shape
  n_edges = edge_weight.shape[0]
  d_reg = d // LANES

  epw = -(-n_edges // NW)
  n_win = -(-epw // W)
  n_win = -(-n_win // (2 * CH)) * (2 * CH)
  e_pad = NW * n_win * W
  pad = e_pad - n_edges
  rows_per_sub = -(-(-(-n_nodes // NS)) // 8) * 8
  n_rows_pad = NS * rows_per_sub

  spread = (jnp.arange(pad, dtype=jnp.int32) * 97) % n_nodes
  col_a = jnp.concatenate([edge_index[1], spread]).reshape(NW, n_win, W)
  row_a = jnp.concatenate([edge_index[0], spread]).reshape(NW, n_win, W)
  w_flat = jnp.concatenate(
      [edge_weight, jnp.zeros((pad,), edge_weight.dtype)]
  ).reshape(NW, n_win, W)

  res, wa_a = pl.pallas_call(
      _prep_body,
      out_shape=(
          jax.ShapeDtypeStruct((n_nodes, d), jnp.float32),
          jax.ShapeDtypeStruct((NW, n_win, W), jnp.float32),
      ),
  )(x, w_flat)

  mesh = plsc.VectorSubcoreMesh(core_axis_name="c", subcore_axis_name="s")
  sc_step = pl.kernel(
      functools.partial(_sc_step_body, n_win, rows_per_sub, d_reg),
      out_type=jax.ShapeDtypeStruct((NC, n_rows_pad, d), jnp.float32),
      mesh=mesh,
      scratch_types=[
          pltpu.VMEM_SHARED((n_rows_pad, d), jnp.float32),
          pltpu.VMEM((n_win, W), jnp.int32),
          pltpu.VMEM((CH, W), jnp.int32),
          pltpu.VMEM((CH, W), jnp.int32),
          pltpu.VMEM((CH, W), jnp.float32),
          pltpu.VMEM((CH, W), jnp.float32),
          pltpu.VMEM((W, d), jnp.float32),
          pltpu.VMEM((W, d), jnp.float32),
          pltpu.SemaphoreType.DMA,
          pltpu.SemaphoreType.DMA,
          pltpu.SemaphoreType.DMA,
          pltpu.SemaphoreType.DMA,
          pltpu.SemaphoreType.DMA,
          pltpu.SemaphoreType.DMA,
      ],
  )

  nrb = 1000
  combine = pl.pallas_call(
      _combine_body,
      grid=(n_nodes // nrb,),
      in_specs=[
          pl.BlockSpec((NC, nrb, d), lambda i: (0, i, 0)),
          pl.BlockSpec((nrb, d), lambda i: (i, 0)),
      ],
      out_specs=pl.BlockSpec((nrb, d), lambda i: (i, 0)),
      out_shape=jax.ShapeDtypeStruct((n_nodes, d), jnp.float32),
  )

  out = x
  for _ in range(ITERS):
    parts = sc_step(out, col_a, row_a, wa_a)
    out = combine(parts, res)
  return out
